# Initial kernel scaffold; baseline (speedup 1.0000x reference)
import functools

import jax
import jax.numpy as jnp
from jax import lax
from jax.experimental import pallas as pl
from jax.experimental.pallas import tpu as pltpu
from jax.experimental.pallas import tpu_sc as plsc

N = 4194304
NR = 65536
G = 4
SUB = 64
ROWS = SUB * 128
NT = 32
CH = N // NT
P = 16384
NP = CH // P


def _scan_body(a_ref, o_ref, lv_ref, w2_ref, off2_ref, s1last_ref, w3run_ref,
               off3_ref, s2last_ref):
    g = pl.program_id(0)

    @pl.when(g == 0)
    def _init():
        zero = jnp.zeros((1, 1), jnp.float32)
        s1last_ref[...] = zero
        w3run_ref[...] = zero
        off3_ref[...] = zero
        s2last_ref[...] = zero

    lv_ref[...] = jnp.log1p(-a_ref[0])

    acc = lv_ref[0]
    o_ref[0, 0] = acc
    for j in range(1, 128):
        acc = acc + lv_ref[j]
        o_ref[0, j] = acc
    s1 = acc

    acc2 = s1[:, 0:1]
    w2_ref[:, 0:1] = acc2
    for l in range(1, 128):
        acc2 = acc2 + s1[:, l:l + 1]
        w2_ref[:, l:l + 1] = acc2
    w2 = w2_ref[...]

    s2col = w2[:, 127:128]
    w3run = w3run_ref[...]
    off3 = off3_ref[...]
    s2last = s2last_ref[...]
    for s in range(SUB):
        off2_ref[s:s + 1, :] = s2last
        w3run = w3run + s2col[s:s + 1, :]
        s2last = w3run + off3
        kg = g * SUB + s
        pred = (kg + 1) % 128 == 0
        off3 = jnp.where(pred, off3 + w3run, off3)
        w3run = jnp.where(pred, jnp.zeros((1, 1), jnp.float32), w3run)
    w3run_ref[...] = w3run
    off3_ref[...] = off3
    s2last_ref[...] = s2last

    s1f = w2 + off2_ref[...]
    lastcol = s1f[:, 127:128]
    shifted = jnp.concatenate([s1last_ref[...], lastcol[:SUB - 1]], axis=0)
    off1 = jnp.concatenate([shifted, s1f[:, :127]], axis=1)
    s1last_ref[...] = lastcol[SUB - 1:SUB]

    for j in range(128):
        o_ref[0, j] = (o_ref[0, j] + off1) - lv_ref[j]


def _tc_scan(a4):
    return pl.pallas_call(
        _scan_body,
        grid=(G,),
        in_specs=[pl.BlockSpec((1, 128, SUB, 128), lambda i: (i, 0, 0, 0))],
        out_specs=pl.BlockSpec((1, 128, SUB, 128), lambda i: (i, 0, 0, 0)),
        out_shape=jax.ShapeDtypeStruct((G, 128, SUB, 128), jnp.float32),
        scratch_shapes=[
            pltpu.VMEM((128, SUB, 128), jnp.float32),
            pltpu.VMEM((SUB, 128), jnp.float32),
            pltpu.VMEM((SUB, 1), jnp.float32),
            pltpu.VMEM((1, 1), jnp.float32),
            pltpu.VMEM((1, 1), jnp.float32),
            pltpu.VMEM((1, 1), jnp.float32),
            pltpu.VMEM((1, 1), jnp.float32),
        ],
    )(a4)


_MESH = plsc.VectorSubcoreMesh(core_axis_name="c", subcore_axis_name="s")


@functools.partial(
    pl.kernel,
    mesh=_MESH,
    out_type=jax.ShapeDtypeStruct((NT, NR), jnp.float32),
    scratch_types=[
        pltpu.VMEM((P + 16,), jnp.int32),
        pltpu.VMEM((P,), jnp.float32),
        pltpu.VMEM((NR,), jnp.float32),
    ],
)
def _sc_boundary(ri_hbm, cs_hbm, tables_hbm, riv, csv, tbl):
    wid = lax.axis_index("s") * 2 + lax.axis_index("c")
    base = wid * CH

    def initb(i, c):
        tbl[pl.ds(i * 16, 16)] = jnp.full((16,), 1.0, jnp.float32)
        return c

    lax.fori_loop(0, NR // 16, initb, 0)

    def piece(p, c):
        gb = base + p * P
        pltpu.sync_copy(ri_hbm.at[pl.ds(gb, P)], riv.at[pl.ds(16, P)])

        @pl.when(gb == 0)
        def _first():
            riv[pl.ds(0, 16)] = jnp.full((16,), -1, jnp.int32)

        @pl.when(gb > 0)
        def _rest():
            pltpu.sync_copy(ri_hbm.at[pl.ds(gb - 16, 16)], riv.at[pl.ds(0, 16)])

        pltpu.sync_copy(cs_hbm.at[pl.ds(gb, P)], csv)

        def inner(j, c2):
            off = j * 16
            cur = riv[pl.ds(off + 16, 16)]
            idxv = lax.iota(jnp.int32, 16) + (off + 15)
            prev = plsc.load_gather(riv, [idxv])
            bnd = cur != prev
            cur = jnp.minimum(cur, NR - 1)
            plsc.store_scatter(tbl, [cur], csv[pl.ds(off, 16)], mask=bnd)
            return c2

        lax.fori_loop(0, P // 16, inner, 0)
        return c

    lax.fori_loop(0, NP, piece, 0)
    pltpu.sync_copy(tbl, tables_hbm.at[wid])


@functools.partial(
    pl.kernel,
    mesh=_MESH,
    out_type=jax.ShapeDtypeStruct((NR,), jnp.float32),
    scratch_types=[pltpu.VMEM((NT, NR // NT), jnp.float32)],
)
def _sc_merge(tables_hbm, table_hbm, stg):
    wid = lax.axis_index("s") * 2 + lax.axis_index("c")
    s0 = wid * (NR // NT)
    for t in range(NT):
        pltpu.sync_copy(tables_hbm.at[t, pl.ds(s0, NR // NT)], stg.at[t])

    def mj(j, c):
        off = j * 16
        acc = stg[0, pl.ds(off, 16)]
        for t in range(1, NT):
            acc = jnp.minimum(acc, stg[t, pl.ds(off, 16)])
        stg[0, pl.ds(off, 16)] = acc
        return c

    lax.fori_loop(0, NR // NT // 16, mj, 0)
    pltpu.sync_copy(stg.at[0], table_hbm.at[pl.ds(s0, NR // NT)])


@functools.partial(
    pl.kernel,
    mesh=_MESH,
    out_type=jax.ShapeDtypeStruct((N,), jnp.float32),
    scratch_types=[
        pltpu.VMEM((NR,), jnp.float32),
        pltpu.VMEM((P,), jnp.int32),
        pltpu.VMEM((P,), jnp.float32),
        pltpu.VMEM((P,), jnp.float32),
    ],
)
def _sc_gather(table_hbm, ri_hbm, cs_hbm, out_hbm, tbl, riv, csv, ov):
    wid = lax.axis_index("s") * 2 + lax.axis_index("c")
    base = wid * CH
    pltpu.sync_copy(table_hbm, tbl)

    def piece(p, c):
        gb = base + p * P
        pltpu.sync_copy(ri_hbm.at[pl.ds(gb, P)], riv)
        pltpu.sync_copy(cs_hbm.at[pl.ds(gb, P)], csv)

        def inner(j, c2):
            off = j * 16
            idx = jnp.minimum(riv[pl.ds(off, 16)], NR - 1)
            bv = plsc.load_gather(tbl, [idx])
            ov[pl.ds(off, 16)] = jnp.exp(csv[pl.ds(off, 16)] - bv)
            return c2

        lax.fori_loop(0, P // 16, inner, 0)
        pltpu.sync_copy(ov, out_hbm.at[pl.ds(gb, P)])
        return c

    lax.fori_loop(0, NP, piece, 0)


def kernel(alphas, ray_indices, n_rays):
    a4 = alphas.reshape(G, ROWS, 128).transpose(0, 2, 1).reshape(G, 128, SUB, 128)
    cs4 = _tc_scan(a4)
    cs = cs4.reshape(G, 128, ROWS).transpose(0, 2, 1).reshape(-1)
    tables = _sc_boundary(ray_indices, cs)
    table = _sc_merge(tables)
    return _sc_gather(table, ray_indices, cs)


# speedup vs baseline: 102.4151x; 102.4151x over previous
import functools

import jax
import jax.numpy as jnp
from jax import lax
from jax.experimental import pallas as pl
from jax.experimental.pallas import tpu as pltpu
from jax.experimental.pallas import tpu_sc as plsc

N = 4194304
NR = 65536
G = 4
SUB = 64
ROWS = SUB * 128
NT = 32
CH = N // NT
P = 16384
NP = CH // P


def _scan_body(a_ref, o_ref, lv_ref, w2_ref, off2_ref, s1last_ref, w3run_ref,
               off3_ref, s2last_ref):
    g = pl.program_id(0)

    @pl.when(g == 0)
    def _init():
        zero = jnp.zeros((1, 1), jnp.float32)
        s1last_ref[...] = zero
        w3run_ref[...] = zero
        off3_ref[...] = zero
        s2last_ref[...] = zero

    lv_ref[...] = jnp.log1p(-a_ref[0])

    acc = lv_ref[0]
    o_ref[0, 0] = acc
    for j in range(1, 128):
        acc = acc + lv_ref[j]
        o_ref[0, j] = acc
    s1 = acc

    acc2 = s1[:, 0:1]
    w2_ref[:, 0:1] = acc2
    for l in range(1, 128):
        acc2 = acc2 + s1[:, l:l + 1]
        w2_ref[:, l:l + 1] = acc2
    w2 = w2_ref[...]

    s2col = w2[:, 127:128]
    w3run = w3run_ref[...]
    off3 = off3_ref[...]
    s2last = s2last_ref[...]
    for s in range(SUB):
        off2_ref[s:s + 1, :] = s2last
        w3run = w3run + s2col[s:s + 1, :]
        s2last = w3run + off3
        kg = g * SUB + s
        pred = (kg + 1) % 128 == 0
        off3 = jnp.where(pred, off3 + w3run, off3)
        w3run = jnp.where(pred, jnp.zeros((1, 1), jnp.float32), w3run)
    w3run_ref[...] = w3run
    off3_ref[...] = off3
    s2last_ref[...] = s2last

    s1f = w2 + off2_ref[...]
    lastcol = s1f[:, 127:128]
    shifted = jnp.concatenate([s1last_ref[...], lastcol[:SUB - 1]], axis=0)
    off1 = jnp.concatenate([shifted, s1f[:, :127]], axis=1)
    s1last_ref[...] = lastcol[SUB - 1:SUB]

    for j in range(128):
        o_ref[0, j] = (o_ref[0, j] + off1) - lv_ref[j]


def _tc_scan(a4):
    return pl.pallas_call(
        _scan_body,
        grid=(G,),
        in_specs=[pl.BlockSpec((1, 128, SUB, 128), lambda i: (i, 0, 0, 0))],
        out_specs=pl.BlockSpec((1, 128, SUB, 128), lambda i: (i, 0, 0, 0)),
        out_shape=jax.ShapeDtypeStruct((G, 128, SUB, 128), jnp.float32),
        scratch_shapes=[
            pltpu.VMEM((128, SUB, 128), jnp.float32),
            pltpu.VMEM((SUB, 128), jnp.float32),
            pltpu.VMEM((SUB, 1), jnp.float32),
            pltpu.VMEM((1, 1), jnp.float32),
            pltpu.VMEM((1, 1), jnp.float32),
            pltpu.VMEM((1, 1), jnp.float32),
            pltpu.VMEM((1, 1), jnp.float32),
        ],
    )(a4)


_MESH = plsc.VectorSubcoreMesh(core_axis_name="c", subcore_axis_name="s")


@functools.partial(
    pl.kernel,
    mesh=_MESH,
    compiler_params=pltpu.CompilerParams(needs_layout_passes=False),
    out_type=jax.ShapeDtypeStruct((NT, NR), jnp.float32),
    scratch_types=[
        pltpu.VMEM((P + 16,), jnp.int32),
        pltpu.VMEM((P,), jnp.float32),
        pltpu.VMEM((NR,), jnp.float32),
    ],
)
def _sc_boundary(ri_hbm, cs_hbm, tables_hbm, riv, csv, tbl):
    wid = lax.axis_index("s") * 2 + lax.axis_index("c")
    base = wid * CH

    def initb(i, c):
        tbl[pl.ds(i * 16, 16)] = jnp.full((16,), 1.0, jnp.float32)
        return c

    lax.fori_loop(0, NR // 16, initb, 0)

    def piece(p, c):
        gb = base + p * P
        pltpu.sync_copy(ri_hbm.at[pl.ds(gb, P)], riv.at[pl.ds(16, P)])

        @pl.when(gb == 0)
        def _first():
            riv[pl.ds(0, 16)] = jnp.full((16,), -1, jnp.int32)

        @pl.when(gb > 0)
        def _rest():
            pltpu.sync_copy(ri_hbm.at[pl.ds(gb - 16, 16)], riv.at[pl.ds(0, 16)])

        pltpu.sync_copy(cs_hbm.at[pl.ds(gb, P)], csv)

        def inner(j, c2):
            off = j * 16
            cur = riv[pl.ds(off + 16, 16)]
            idxv = lax.iota(jnp.int32, 16) + (off + 15)
            prev = plsc.load_gather(riv, [idxv])
            bnd = cur != prev
            cur = jnp.minimum(cur, NR - 1)
            plsc.store_scatter(tbl, [cur], csv[pl.ds(off, 16)], mask=bnd)
            return c2

        lax.fori_loop(0, P // 16, inner, 0)
        return c

    lax.fori_loop(0, NP, piece, 0)
    pltpu.sync_copy(tbl, tables_hbm.at[wid])


@functools.partial(
    pl.kernel,
    mesh=_MESH,
    compiler_params=pltpu.CompilerParams(needs_layout_passes=False),
    out_type=jax.ShapeDtypeStruct((NR,), jnp.float32),
    scratch_types=[pltpu.VMEM((NT, NR // NT), jnp.float32)],
)
def _sc_merge(tables_hbm, table_hbm, stg):
    wid = lax.axis_index("s") * 2 + lax.axis_index("c")
    s0 = wid * (NR // NT)
    for t in range(NT):
        pltpu.sync_copy(tables_hbm.at[t, pl.ds(s0, NR // NT)], stg.at[t])

    def mj(j, c):
        off = j * 16
        acc = stg[0, pl.ds(off, 16)]
        for t in range(1, NT):
            acc = jnp.minimum(acc, stg[t, pl.ds(off, 16)])
        stg[0, pl.ds(off, 16)] = acc
        return c

    lax.fori_loop(0, NR // NT // 16, mj, 0)
    pltpu.sync_copy(stg.at[0], table_hbm.at[pl.ds(s0, NR // NT)])


@functools.partial(
    pl.kernel,
    mesh=_MESH,
    compiler_params=pltpu.CompilerParams(needs_layout_passes=False),
    out_type=jax.ShapeDtypeStruct((N,), jnp.float32),
    scratch_types=[
        pltpu.VMEM((NR,), jnp.float32),
        pltpu.VMEM((P,), jnp.int32),
        pltpu.VMEM((P,), jnp.float32),
        pltpu.VMEM((P,), jnp.float32),
    ],
)
def _sc_gather(table_hbm, ri_hbm, cs_hbm, out_hbm, tbl, riv, csv, ov):
    wid = lax.axis_index("s") * 2 + lax.axis_index("c")
    base = wid * CH
    pltpu.sync_copy(table_hbm, tbl)

    def piece(p, c):
        gb = base + p * P
        pltpu.sync_copy(ri_hbm.at[pl.ds(gb, P)], riv)
        pltpu.sync_copy(cs_hbm.at[pl.ds(gb, P)], csv)

        def inner(j, c2):
            off = j * 16
            idx = jnp.minimum(riv[pl.ds(off, 16)], NR - 1)
            bv = plsc.load_gather(tbl, [idx])
            ov[pl.ds(off, 16)] = jnp.exp(csv[pl.ds(off, 16)] - bv)
            return c2

        lax.fori_loop(0, P // 16, inner, 0)
        pltpu.sync_copy(ov, out_hbm.at[pl.ds(gb, P)])
        return c

    lax.fori_loop(0, NP, piece, 0)


def kernel(alphas, ray_indices, n_rays):
    a4 = alphas.reshape(G, ROWS, 128).transpose(0, 2, 1).reshape(G, 128, SUB, 128)
    cs4 = _tc_scan(a4)
    cs = cs4.reshape(G, 128, ROWS).transpose(0, 2, 1).reshape(-1)
    tables = _sc_boundary(ray_indices, cs)
    table = _sc_merge(tables)
    return _sc_gather(table, ray_indices, cs)


# speedup vs baseline: 170.0744x; 1.6606x over previous
import functools

import jax
import jax.numpy as jnp
from jax import lax
from jax.experimental import pallas as pl
from jax.experimental.pallas import tpu as pltpu
from jax.experimental.pallas import tpu_sc as plsc

N = 4194304
NR = 65536
G = 4
SUB = 64
ROWS = SUB * 128
NT = 32
CH = N // NT
P = 16384
NP = CH // P


def _scan_body(a_ref, o_ref, lv_ref, w2_ref, off2_ref, s1last_ref, w3run_ref,
               off3_ref, s2last_ref):
    g = pl.program_id(0)

    @pl.when(g == 0)
    def _init():
        zero = jnp.zeros((1, 1), jnp.float32)
        s1last_ref[...] = zero
        w3run_ref[...] = zero
        off3_ref[...] = zero
        s2last_ref[...] = zero

    lv_ref[...] = jnp.log1p(-a_ref[0])

    acc = lv_ref[0]
    o_ref[0, 0] = acc
    for j in range(1, 128):
        acc = acc + lv_ref[j]
        o_ref[0, j] = acc
    s1 = acc

    acc2 = s1[:, 0:1]
    w2_ref[:, 0:1] = acc2
    for l in range(1, 128):
        acc2 = acc2 + s1[:, l:l + 1]
        w2_ref[:, l:l + 1] = acc2
    w2 = w2_ref[...]

    s2col = w2[:, 127:128]
    w3run = w3run_ref[...]
    off3 = off3_ref[...]
    s2last = s2last_ref[...]
    for s in range(SUB):
        off2_ref[s:s + 1, :] = s2last
        w3run = w3run + s2col[s:s + 1, :]
        s2last = w3run + off3
        kg = g * SUB + s
        pred = (kg + 1) % 128 == 0
        off3 = jnp.where(pred, off3 + w3run, off3)
        w3run = jnp.where(pred, jnp.zeros((1, 1), jnp.float32), w3run)
    w3run_ref[...] = w3run
    off3_ref[...] = off3
    s2last_ref[...] = s2last

    s1f = w2 + off2_ref[...]
    lastcol = s1f[:, 127:128]
    shifted = jnp.concatenate([s1last_ref[...], lastcol[:SUB - 1]], axis=0)
    off1 = jnp.concatenate([shifted, s1f[:, :127]], axis=1)
    s1last_ref[...] = lastcol[SUB - 1:SUB]

    for j in range(128):
        o_ref[0, j] = (o_ref[0, j] + off1) - lv_ref[j]


def _tc_scan(a4):
    return pl.pallas_call(
        _scan_body,
        grid=(G,),
        in_specs=[pl.BlockSpec((1, 128, SUB, 128), lambda i: (i, 0, 0, 0))],
        out_specs=pl.BlockSpec((1, 128, SUB, 128), lambda i: (i, 0, 0, 0)),
        out_shape=jax.ShapeDtypeStruct((G, 128, SUB, 128), jnp.float32),
        scratch_shapes=[
            pltpu.VMEM((128, SUB, 128), jnp.float32),
            pltpu.VMEM((SUB, 128), jnp.float32),
            pltpu.VMEM((SUB, 1), jnp.float32),
            pltpu.VMEM((1, 1), jnp.float32),
            pltpu.VMEM((1, 1), jnp.float32),
            pltpu.VMEM((1, 1), jnp.float32),
            pltpu.VMEM((1, 1), jnp.float32),
        ],
    )(a4)


_MESH = plsc.VectorSubcoreMesh(core_axis_name="c", subcore_axis_name="s")


@functools.partial(
    pl.kernel,
    mesh=_MESH,
    compiler_params=pltpu.CompilerParams(needs_layout_passes=False),
    out_type=jax.ShapeDtypeStruct((NT, NR), jnp.float32),
    scratch_types=[
        pltpu.VMEM((P + 16,), jnp.int32),
        pltpu.VMEM((P,), jnp.float32),
        pltpu.VMEM((NR,), jnp.float32),
    ],
)
def _sc_boundary(ri_hbm, cs_hbm, tables_hbm, riv, csv, tbl):
    wid = lax.axis_index("s") * 2 + lax.axis_index("c")
    base = wid * CH

    @plsc.parallel_loop(0, NR // 16, 1, unroll=8)
    def _initb(i):
        tbl[pl.ds(i * 16, 16)] = jnp.full((16,), 1.0, jnp.float32)

    def piece(p, c):
        gb = base + p * P
        pltpu.sync_copy(ri_hbm.at[pl.ds(gb, P)], riv.at[pl.ds(16, P)])

        @pl.when(gb == 0)
        def _first():
            riv[pl.ds(0, 16)] = jnp.full((16,), -1, jnp.int32)

        @pl.when(gb > 0)
        def _rest():
            pltpu.sync_copy(ri_hbm.at[pl.ds(gb - 16, 16)], riv.at[pl.ds(0, 16)])

        pltpu.sync_copy(cs_hbm.at[pl.ds(gb, P)], csv)

        @plsc.parallel_loop(0, P // 16, 1, unroll=8)
        def _inner(j):
            off = j * 16
            cur = riv[pl.ds(off + 16, 16)]
            idxv = lax.iota(jnp.int32, 16) + (off + 15)
            prev = plsc.load_gather(riv, [idxv])
            bnd = cur != prev
            cur = jnp.minimum(cur, NR - 1)
            plsc.store_scatter(tbl, [cur], csv[pl.ds(off, 16)], mask=bnd)

        return c

    lax.fori_loop(0, NP, piece, 0)
    pltpu.sync_copy(tbl, tables_hbm.at[wid])


@functools.partial(
    pl.kernel,
    mesh=_MESH,
    compiler_params=pltpu.CompilerParams(needs_layout_passes=False),
    out_type=jax.ShapeDtypeStruct((NR,), jnp.float32),
    scratch_types=[pltpu.VMEM((NT, NR // NT), jnp.float32)],
)
def _sc_merge(tables_hbm, table_hbm, stg):
    wid = lax.axis_index("s") * 2 + lax.axis_index("c")
    s0 = wid * (NR // NT)
    for t in range(NT):
        pltpu.sync_copy(tables_hbm.at[t, pl.ds(s0, NR // NT)], stg.at[t])

    @plsc.parallel_loop(0, NR // NT // 16, 1, unroll=2)
    def _mj(j):
        off = j * 16
        acc = stg[0, pl.ds(off, 16)]
        for t in range(1, NT):
            acc = jnp.minimum(acc, stg[t, pl.ds(off, 16)])
        stg[0, pl.ds(off, 16)] = acc
    pltpu.sync_copy(stg.at[0], table_hbm.at[pl.ds(s0, NR // NT)])


@functools.partial(
    pl.kernel,
    mesh=_MESH,
    compiler_params=pltpu.CompilerParams(needs_layout_passes=False),
    out_type=jax.ShapeDtypeStruct((N,), jnp.float32),
    scratch_types=[
        pltpu.VMEM((NR,), jnp.float32),
        pltpu.VMEM((P,), jnp.int32),
        pltpu.VMEM((P,), jnp.float32),
        pltpu.VMEM((P,), jnp.float32),
    ],
)
def _sc_gather(table_hbm, ri_hbm, cs_hbm, out_hbm, tbl, riv, csv, ov):
    wid = lax.axis_index("s") * 2 + lax.axis_index("c")
    base = wid * CH
    pltpu.sync_copy(table_hbm, tbl)

    def piece(p, c):
        gb = base + p * P
        pltpu.sync_copy(ri_hbm.at[pl.ds(gb, P)], riv)
        pltpu.sync_copy(cs_hbm.at[pl.ds(gb, P)], csv)

        @plsc.parallel_loop(0, P // 16, 1, unroll=8)
        def _inner(j):
            off = j * 16
            idx = jnp.minimum(riv[pl.ds(off, 16)], NR - 1)
            bv = plsc.load_gather(tbl, [idx])
            ov[pl.ds(off, 16)] = jnp.exp(csv[pl.ds(off, 16)] - bv)

        pltpu.sync_copy(ov, out_hbm.at[pl.ds(gb, P)])
        return c

    lax.fori_loop(0, NP, piece, 0)


def kernel(alphas, ray_indices, n_rays):
    a4 = alphas.reshape(G, ROWS, 128).transpose(0, 2, 1).reshape(G, 128, SUB, 128)
    cs4 = _tc_scan(a4)
    cs = cs4.reshape(G, 128, ROWS).transpose(0, 2, 1).reshape(-1)
    tables = _sc_boundary(ray_indices, cs)
    table = _sc_merge(tables)
    return _sc_gather(table, ray_indices, cs)


# speedup vs baseline: 173.1198x; 1.0179x over previous
import functools

import jax
import jax.numpy as jnp
from jax import lax
from jax.experimental import pallas as pl
from jax.experimental.pallas import tpu as pltpu
from jax.experimental.pallas import tpu_sc as plsc

N = 4194304
NR = 65536
G = 4
SUB = 64
ROWS = SUB * 128
NT = 32
CH = N // NT
P = 16384
NP = CH // P


def _scan_body(a_ref, o_ref, lv_ref, w1_ref, w2_ref, off2_ref, s1last_ref,
               w3run_ref, off3_ref, s2last_ref):
    g = pl.program_id(0)

    @pl.when(g == 0)
    def _init():
        zero = jnp.zeros((1, 1), jnp.float32)
        s1last_ref[...] = zero
        w3run_ref[...] = zero
        off3_ref[...] = zero
        s2last_ref[...] = zero

    for s in range(SUB):
        chunk = a_ref[0, s * 128:(s + 1) * 128, :]
        lv_ref[:, s, :] = jnp.log1p(-chunk).T

    acc = lv_ref[0]
    w1_ref[0] = acc
    for j in range(1, 128):
        acc = acc + lv_ref[j]
        w1_ref[j] = acc
    s1 = acc

    acc2 = s1[:, 0:1]
    w2_ref[:, 0:1] = acc2
    for l in range(1, 128):
        acc2 = acc2 + s1[:, l:l + 1]
        w2_ref[:, l:l + 1] = acc2
    w2 = w2_ref[...]

    s2col = w2[:, 127:128]
    w3run = w3run_ref[...]
    off3 = off3_ref[...]
    s2last = s2last_ref[...]
    for s in range(SUB):
        off2_ref[s:s + 1, :] = s2last
        w3run = w3run + s2col[s:s + 1, :]
        s2last = w3run + off3
        kg = g * SUB + s
        pred = (kg + 1) % 128 == 0
        off3 = jnp.where(pred, off3 + w3run, off3)
        w3run = jnp.where(pred, jnp.zeros((1, 1), jnp.float32), w3run)
    w3run_ref[...] = w3run
    off3_ref[...] = off3
    s2last_ref[...] = s2last

    s1f = w2 + off2_ref[...]
    lastcol = s1f[:, 127:128]
    shifted = jnp.concatenate([s1last_ref[...], lastcol[:SUB - 1]], axis=0)
    off1 = jnp.concatenate([shifted, s1f[:, :127]], axis=1)
    s1last_ref[...] = lastcol[SUB - 1:SUB]

    for s in range(SUB):
        cs_chunk = (w1_ref[:, s, :] + off1[s:s + 1, :]) - lv_ref[:, s, :]
        o_ref[0, s * 128:(s + 1) * 128, :] = cs_chunk.T


def _tc_scan(a4):
    return pl.pallas_call(
        _scan_body,
        grid=(G,),
        in_specs=[pl.BlockSpec((1, ROWS, 128), lambda i: (i, 0, 0))],
        out_specs=pl.BlockSpec((1, ROWS, 128), lambda i: (i, 0, 0)),
        out_shape=jax.ShapeDtypeStruct((G, ROWS, 128), jnp.float32),
        scratch_shapes=[
            pltpu.VMEM((128, SUB, 128), jnp.float32),
            pltpu.VMEM((128, SUB, 128), jnp.float32),
            pltpu.VMEM((SUB, 128), jnp.float32),
            pltpu.VMEM((SUB, 1), jnp.float32),
            pltpu.VMEM((1, 1), jnp.float32),
            pltpu.VMEM((1, 1), jnp.float32),
            pltpu.VMEM((1, 1), jnp.float32),
            pltpu.VMEM((1, 1), jnp.float32),
        ],
    )(a4)


_MESH = plsc.VectorSubcoreMesh(core_axis_name="c", subcore_axis_name="s")


@functools.partial(
    pl.kernel,
    mesh=_MESH,
    compiler_params=pltpu.CompilerParams(needs_layout_passes=False),
    out_type=jax.ShapeDtypeStruct((NT, NR), jnp.float32),
    scratch_types=[
        pltpu.VMEM((P + 16,), jnp.int32),
        pltpu.VMEM((P,), jnp.float32),
        pltpu.VMEM((NR,), jnp.float32),
    ],
)
def _sc_boundary(ri_hbm, cs_hbm, tables_hbm, riv, csv, tbl):
    wid = lax.axis_index("s") * 2 + lax.axis_index("c")
    base = wid * CH

    @plsc.parallel_loop(0, NR // 16, 1, unroll=8)
    def _initb(i):
        tbl[pl.ds(i * 16, 16)] = jnp.full((16,), 1.0, jnp.float32)

    def piece(p, c):
        gb = base + p * P
        pltpu.sync_copy(ri_hbm.at[pl.ds(gb, P)], riv.at[pl.ds(16, P)])

        @pl.when(gb == 0)
        def _first():
            riv[pl.ds(0, 16)] = jnp.full((16,), -1, jnp.int32)

        @pl.when(gb > 0)
        def _rest():
            pltpu.sync_copy(ri_hbm.at[pl.ds(gb - 16, 16)], riv.at[pl.ds(0, 16)])

        pltpu.sync_copy(cs_hbm.at[pl.ds(gb, P)], csv)

        @plsc.parallel_loop(0, P // 16, 1, unroll=8)
        def _inner(j):
            off = j * 16
            cur = riv[pl.ds(off + 16, 16)]
            idxv = lax.iota(jnp.int32, 16) + (off + 15)
            prev = plsc.load_gather(riv, [idxv])
            bnd = cur != prev
            cur = jnp.minimum(cur, NR - 1)
            plsc.store_scatter(tbl, [cur], csv[pl.ds(off, 16)], mask=bnd)

        return c

    lax.fori_loop(0, NP, piece, 0)
    pltpu.sync_copy(tbl, tables_hbm.at[wid])


@functools.partial(
    pl.kernel,
    mesh=_MESH,
    compiler_params=pltpu.CompilerParams(needs_layout_passes=False),
    out_type=jax.ShapeDtypeStruct((NR,), jnp.float32),
    scratch_types=[pltpu.VMEM((NT, NR // NT), jnp.float32)],
)
def _sc_merge(tables_hbm, table_hbm, stg):
    wid = lax.axis_index("s") * 2 + lax.axis_index("c")
    s0 = wid * (NR // NT)
    for t in range(NT):
        pltpu.sync_copy(tables_hbm.at[t, pl.ds(s0, NR // NT)], stg.at[t])

    @plsc.parallel_loop(0, NR // NT // 16, 1, unroll=2)
    def _mj(j):
        off = j * 16
        acc = stg[0, pl.ds(off, 16)]
        for t in range(1, NT):
            acc = jnp.minimum(acc, stg[t, pl.ds(off, 16)])
        stg[0, pl.ds(off, 16)] = acc
    pltpu.sync_copy(stg.at[0], table_hbm.at[pl.ds(s0, NR // NT)])


@functools.partial(
    pl.kernel,
    mesh=_MESH,
    compiler_params=pltpu.CompilerParams(needs_layout_passes=False),
    out_type=jax.ShapeDtypeStruct((N,), jnp.float32),
    scratch_types=[
        pltpu.VMEM((NR,), jnp.float32),
        pltpu.VMEM((P,), jnp.int32),
        pltpu.VMEM((P,), jnp.float32),
        pltpu.VMEM((P,), jnp.float32),
    ],
)
def _sc_gather(table_hbm, ri_hbm, cs_hbm, out_hbm, tbl, riv, csv, ov):
    wid = lax.axis_index("s") * 2 + lax.axis_index("c")
    base = wid * CH
    pltpu.sync_copy(table_hbm, tbl)

    def piece(p, c):
        gb = base + p * P
        pltpu.sync_copy(ri_hbm.at[pl.ds(gb, P)], riv)
        pltpu.sync_copy(cs_hbm.at[pl.ds(gb, P)], csv)

        @plsc.parallel_loop(0, P // 16, 1, unroll=8)
        def _inner(j):
            off = j * 16
            idx = jnp.minimum(riv[pl.ds(off, 16)], NR - 1)
            bv = plsc.load_gather(tbl, [idx])
            ov[pl.ds(off, 16)] = jnp.exp(csv[pl.ds(off, 16)] - bv)

        pltpu.sync_copy(ov, out_hbm.at[pl.ds(gb, P)])
        return c

    lax.fori_loop(0, NP, piece, 0)


def kernel(alphas, ray_indices, n_rays):
    a3 = alphas.reshape(G, ROWS, 128)
    cs = _tc_scan(a3).reshape(-1)
    tables = _sc_boundary(ray_indices, cs)
    table = _sc_merge(tables)
    return _sc_gather(table, ray_indices, cs)


# speedup vs baseline: 219.2894x; 1.2667x over previous
import functools

import jax
import jax.numpy as jnp
from jax import lax
from jax.experimental import pallas as pl
from jax.experimental.pallas import tpu as pltpu
from jax.experimental.pallas import tpu_sc as plsc

N = 4194304
NR = 65536
G = 2
SUB = 128
ROWS = SUB * 128
NT = 32
CH = N // NT
P = 16384
NP = CH // P


def _scan_body(a_ref, o_ref, lv_ref, w1_ref, w2_ref, off2_ref, s1last_ref,
               w3run_ref, off3_ref, s2last_ref):
    g = pl.program_id(0)

    @pl.when(g == 0)
    def _init():
        zero = jnp.zeros((1, 1), jnp.float32)
        s1last_ref[...] = zero
        w3run_ref[...] = zero
        off3_ref[...] = zero
        s2last_ref[...] = zero

    lv_ref[...] = jnp.log1p(-a_ref[0]).T.reshape(128, SUB, 128)

    acc = lv_ref[0]
    w1_ref[0] = acc
    for j in range(1, 128):
        acc = acc + lv_ref[j]
        w1_ref[j] = acc
    s1 = acc

    acc2 = s1[:, 0:1]
    w2_ref[:, 0:1] = acc2
    for l in range(1, 128):
        acc2 = acc2 + s1[:, l:l + 1]
        w2_ref[:, l:l + 1] = acc2
    w2 = w2_ref[...]

    s2col = w2[:, 127:128]
    w3run = w3run_ref[...]
    off3 = off3_ref[...]
    s2last = s2last_ref[...]
    for s in range(SUB):
        off2_ref[s:s + 1, :] = s2last
        w3run = w3run + s2col[s:s + 1, :]
        s2last = w3run + off3
        kg = g * SUB + s
        pred = (kg + 1) % 128 == 0
        off3 = jnp.where(pred, off3 + w3run, off3)
        w3run = jnp.where(pred, jnp.zeros((1, 1), jnp.float32), w3run)
    w3run_ref[...] = w3run
    off3_ref[...] = off3
    s2last_ref[...] = s2last

    s1f = w2 + off2_ref[...]
    lastcol = s1f[:, 127:128]
    shifted = jnp.concatenate([s1last_ref[...], lastcol[:SUB - 1]], axis=0)
    off1 = jnp.concatenate([shifted, s1f[:, :127]], axis=1)
    s1last_ref[...] = lastcol[SUB - 1:SUB]

    cs3 = (w1_ref[...] + off1[None]) - lv_ref[...]
    o_ref[0] = cs3.reshape(128, ROWS).T


def _tc_scan(a4):
    return pl.pallas_call(
        _scan_body,
        grid=(G,),
        in_specs=[pl.BlockSpec((1, ROWS, 128), lambda i: (i, 0, 0))],
        out_specs=pl.BlockSpec((1, ROWS, 128), lambda i: (i, 0, 0)),
        out_shape=jax.ShapeDtypeStruct((G, ROWS, 128), jnp.float32),
        scratch_shapes=[
            pltpu.VMEM((128, SUB, 128), jnp.float32),
            pltpu.VMEM((128, SUB, 128), jnp.float32),
            pltpu.VMEM((SUB, 128), jnp.float32),
            pltpu.VMEM((SUB, 1), jnp.float32),
            pltpu.VMEM((1, 1), jnp.float32),
            pltpu.VMEM((1, 1), jnp.float32),
            pltpu.VMEM((1, 1), jnp.float32),
            pltpu.VMEM((1, 1), jnp.float32),
        ],
    )(a4)


_MESH = plsc.VectorSubcoreMesh(core_axis_name="c", subcore_axis_name="s")


@functools.partial(
    pl.kernel,
    mesh=_MESH,
    compiler_params=pltpu.CompilerParams(needs_layout_passes=False),
    out_type=jax.ShapeDtypeStruct((NT, NR), jnp.float32),
    scratch_types=[
        pltpu.VMEM((P + 16,), jnp.int32),
        pltpu.VMEM((P,), jnp.float32),
        pltpu.VMEM((NR,), jnp.float32),
    ],
)
def _sc_boundary(ri_hbm, cs_hbm, tables_hbm, riv, csv, tbl):
    wid = lax.axis_index("s") * 2 + lax.axis_index("c")
    base = wid * CH

    @plsc.parallel_loop(0, NR // 16, 1, unroll=8)
    def _initb(i):
        tbl[pl.ds(i * 16, 16)] = jnp.full((16,), 1.0, jnp.float32)

    def piece(p, c):
        gb = base + p * P
        pltpu.sync_copy(ri_hbm.at[pl.ds(gb, P)], riv.at[pl.ds(16, P)])

        @pl.when(gb == 0)
        def _first():
            riv[pl.ds(0, 16)] = jnp.full((16,), -1, jnp.int32)

        @pl.when(gb > 0)
        def _rest():
            pltpu.sync_copy(ri_hbm.at[pl.ds(gb - 16, 16)], riv.at[pl.ds(0, 16)])

        pltpu.sync_copy(cs_hbm.at[pl.ds(gb, P)], csv)

        @plsc.parallel_loop(0, P // 16, 1, unroll=8)
        def _inner(j):
            off = j * 16
            cur = riv[pl.ds(off + 16, 16)]
            idxv = lax.iota(jnp.int32, 16) + (off + 15)
            prev = plsc.load_gather(riv, [idxv])
            bnd = cur != prev
            cur = jnp.minimum(cur, NR - 1)
            plsc.store_scatter(tbl, [cur], csv[pl.ds(off, 16)], mask=bnd)

        return c

    lax.fori_loop(0, NP, piece, 0)
    pltpu.sync_copy(tbl, tables_hbm.at[wid])


@functools.partial(
    pl.kernel,
    mesh=_MESH,
    compiler_params=pltpu.CompilerParams(needs_layout_passes=False),
    out_type=jax.ShapeDtypeStruct((NR,), jnp.float32),
    scratch_types=[
        pltpu.VMEM((NT, NR // NT), jnp.float32),
        pltpu.SemaphoreType.DMA,
    ],
)
def _sc_merge(tables_hbm, table_hbm, stg, sem):
    wid = lax.axis_index("s") * 2 + lax.axis_index("c")
    s0 = wid * (NR // NT)
    copies = [
        pltpu.make_async_copy(tables_hbm.at[t, pl.ds(s0, NR // NT)], stg.at[t], sem)
        for t in range(NT)
    ]
    for c in copies:
        c.start()
    for c in copies:
        c.wait()

    @plsc.parallel_loop(0, NR // NT // 16, 1, unroll=4)
    def _mj(j):
        off = j * 16
        acc = stg[0, pl.ds(off, 16)]
        for t in range(1, NT):
            acc = jnp.minimum(acc, stg[t, pl.ds(off, 16)])
        stg[0, pl.ds(off, 16)] = acc
    pltpu.sync_copy(stg.at[0], table_hbm.at[pl.ds(s0, NR // NT)])


@functools.partial(
    pl.kernel,
    mesh=_MESH,
    compiler_params=pltpu.CompilerParams(needs_layout_passes=False),
    out_type=jax.ShapeDtypeStruct((N,), jnp.float32),
    scratch_types=[
        pltpu.VMEM((NR,), jnp.float32),
        pltpu.VMEM((P,), jnp.int32),
        pltpu.VMEM((P,), jnp.float32),
        pltpu.VMEM((P,), jnp.float32),
    ],
)
def _sc_gather(table_hbm, ri_hbm, cs_hbm, out_hbm, tbl, riv, csv, ov):
    wid = lax.axis_index("s") * 2 + lax.axis_index("c")
    base = wid * CH
    pltpu.sync_copy(table_hbm, tbl)

    def piece(p, c):
        gb = base + p * P
        pltpu.sync_copy(ri_hbm.at[pl.ds(gb, P)], riv)
        pltpu.sync_copy(cs_hbm.at[pl.ds(gb, P)], csv)

        @plsc.parallel_loop(0, P // 16, 1, unroll=8)
        def _inner(j):
            off = j * 16
            idx = jnp.minimum(riv[pl.ds(off, 16)], NR - 1)
            bv = plsc.load_gather(tbl, [idx])
            ov[pl.ds(off, 16)] = jnp.exp(csv[pl.ds(off, 16)] - bv)

        pltpu.sync_copy(ov, out_hbm.at[pl.ds(gb, P)])
        return c

    lax.fori_loop(0, NP, piece, 0)


def kernel(alphas, ray_indices, n_rays):
    a3 = alphas.reshape(G, ROWS, 128)
    cs = _tc_scan(a3).reshape(-1)
    tables = _sc_boundary(ray_indices, cs)
    table = _sc_merge(tables)
    return _sc_gather(table, ray_indices, cs)


# speedup vs baseline: 250.7352x; 1.1434x over previous
import functools

import jax
import jax.numpy as jnp
from jax import lax
from jax.experimental import pallas as pl
from jax.experimental.pallas import tpu as pltpu
from jax.experimental.pallas import tpu_sc as plsc

N = 4194304
NR = 65536
G = 2
SUB = 128
ROWS = SUB * 128
NT = 32
CH = N // NT
P = 16384
NP = CH // P


def _scan_body(a_ref, o_ref, lv_ref, w1_ref, w2_ref, off2_ref, s1last_ref,
               w3run_ref, off3_ref, s2last_ref):
    g = pl.program_id(0)

    @pl.when(g == 0)
    def _init():
        zero = jnp.zeros((1, 1), jnp.float32)
        s1last_ref[...] = zero
        w3run_ref[...] = zero
        off3_ref[...] = zero
        s2last_ref[...] = zero

    lv_ref[...] = jnp.log1p(-a_ref[0]).T.reshape(128, SUB, 128)

    acc = lv_ref[0]
    w1_ref[0] = acc
    for j in range(1, 128):
        acc = acc + lv_ref[j]
        w1_ref[j] = acc
    s1 = acc

    acc2 = s1[:, 0:1]
    w2_ref[:, 0:1] = acc2
    for l in range(1, 128):
        acc2 = acc2 + s1[:, l:l + 1]
        w2_ref[:, l:l + 1] = acc2
    w2 = w2_ref[...]

    s2col = w2[:, 127:128]
    w3run = w3run_ref[...]
    off3 = off3_ref[...]
    s2last = s2last_ref[...]
    for s in range(SUB):
        off2_ref[s:s + 1, :] = s2last
        w3run = w3run + s2col[s:s + 1, :]
        s2last = w3run + off3
        kg = g * SUB + s
        pred = (kg + 1) % 128 == 0
        off3 = jnp.where(pred, off3 + w3run, off3)
        w3run = jnp.where(pred, jnp.zeros((1, 1), jnp.float32), w3run)
    w3run_ref[...] = w3run
    off3_ref[...] = off3
    s2last_ref[...] = s2last

    s1f = w2 + off2_ref[...]
    lastcol = s1f[:, 127:128]
    shifted = jnp.concatenate([s1last_ref[...], lastcol[:SUB - 1]], axis=0)
    off1 = jnp.concatenate([shifted, s1f[:, :127]], axis=1)
    s1last_ref[...] = lastcol[SUB - 1:SUB]

    cs3 = (w1_ref[...] + off1[None]) - lv_ref[...]
    o_ref[0] = cs3.reshape(128, ROWS).T


def _tc_scan(a4):
    return pl.pallas_call(
        _scan_body,
        grid=(G,),
        in_specs=[pl.BlockSpec((1, ROWS, 128), lambda i: (i, 0, 0))],
        out_specs=pl.BlockSpec((1, ROWS, 128), lambda i: (i, 0, 0)),
        out_shape=jax.ShapeDtypeStruct((G, ROWS, 128), jnp.float32),
        scratch_shapes=[
            pltpu.VMEM((128, SUB, 128), jnp.float32),
            pltpu.VMEM((128, SUB, 128), jnp.float32),
            pltpu.VMEM((SUB, 128), jnp.float32),
            pltpu.VMEM((SUB, 1), jnp.float32),
            pltpu.VMEM((1, 1), jnp.float32),
            pltpu.VMEM((1, 1), jnp.float32),
            pltpu.VMEM((1, 1), jnp.float32),
            pltpu.VMEM((1, 1), jnp.float32),
        ],
    )(a4)


_MESH = plsc.VectorSubcoreMesh(core_axis_name="c", subcore_axis_name="s")


@functools.partial(
    pl.kernel,
    mesh=_MESH,
    compiler_params=pltpu.CompilerParams(needs_layout_passes=False),
    out_type=jax.ShapeDtypeStruct((NT, NR), jnp.int32),
    scratch_types=[
        pltpu.VMEM((P + 16,), jnp.int32),
        pltpu.VMEM((NR,), jnp.int32),
    ],
)
def _sc_boundary(ri_hbm, tables_hbm, riv, tbl):
    wid = lax.axis_index("s") * 2 + lax.axis_index("c")
    base = wid * CH

    @plsc.parallel_loop(0, NR // 16, 1, unroll=8)
    def _initb(i):
        tbl[pl.ds(i * 16, 16)] = jnp.full((16,), N - 1, jnp.int32)

    def piece(p, c):
        gb = base + p * P
        pltpu.sync_copy(ri_hbm.at[pl.ds(gb, P)], riv.at[pl.ds(16, P)])

        @pl.when(gb == 0)
        def _first():
            riv[pl.ds(0, 16)] = jnp.full((16,), -1, jnp.int32)

        @pl.when(gb > 0)
        def _rest():
            pltpu.sync_copy(ri_hbm.at[pl.ds(gb - 16, 16)], riv.at[pl.ds(0, 16)])

        @plsc.parallel_loop(0, P // 16, 1, unroll=8)
        def _inner(j):
            off = j * 16
            cur = riv[pl.ds(off + 16, 16)]
            idxv = lax.iota(jnp.int32, 16) + (off + 15)
            prev = plsc.load_gather(riv, [idxv])
            bnd = cur != prev
            cur = jnp.minimum(cur, NR - 1)
            pos = lax.iota(jnp.int32, 16) + (gb + off)
            plsc.store_scatter(tbl, [cur], pos, mask=bnd)

        return c

    lax.fori_loop(0, NP, piece, 0)
    pltpu.sync_copy(tbl, tables_hbm.at[wid])


@functools.partial(
    pl.kernel,
    mesh=_MESH,
    compiler_params=pltpu.CompilerParams(needs_layout_passes=False),
    out_type=jax.ShapeDtypeStruct((NR,), jnp.int32),
    scratch_types=[
        pltpu.VMEM((NT, NR // NT), jnp.int32),
        pltpu.SemaphoreType.DMA,
    ],
)
def _sc_merge(tables_hbm, table_hbm, stg, sem):
    wid = lax.axis_index("s") * 2 + lax.axis_index("c")
    s0 = wid * (NR // NT)
    copies = [
        pltpu.make_async_copy(tables_hbm.at[t, pl.ds(s0, NR // NT)], stg.at[t], sem)
        for t in range(NT)
    ]
    for c in copies:
        c.start()
    for c in copies:
        c.wait()

    @plsc.parallel_loop(0, NR // NT // 16, 1, unroll=4)
    def _mj(j):
        off = j * 16
        acc = stg[0, pl.ds(off, 16)]
        for t in range(1, NT):
            acc = jnp.minimum(acc, stg[t, pl.ds(off, 16)])
        stg[0, pl.ds(off, 16)] = acc
    pltpu.sync_copy(stg.at[0], table_hbm.at[pl.ds(s0, NR // NT)])


@functools.partial(
    pl.kernel,
    mesh=_MESH,
    compiler_params=pltpu.CompilerParams(needs_layout_passes=False),
    out_type=jax.ShapeDtypeStruct((NR,), jnp.float32),
    scratch_types=[
        pltpu.VMEM((NR // NT,), jnp.int32),
        pltpu.VMEM((NR // NT,), jnp.float32),
        pltpu.SemaphoreType.DMA,
    ],
)
def _sc_posval(pos_hbm, cs_hbm, table_hbm, posv, valv, sem):
    wid = lax.axis_index("s") * 2 + lax.axis_index("c")
    s0 = wid * (NR // NT)
    pltpu.sync_copy(pos_hbm.at[pl.ds(s0, NR // NT)], posv)
    pltpu.async_copy(cs_hbm.at[posv], valv, sem).wait()
    pltpu.sync_copy(valv, table_hbm.at[pl.ds(s0, NR // NT)])


@functools.partial(
    pl.kernel,
    mesh=_MESH,
    compiler_params=pltpu.CompilerParams(needs_layout_passes=False),
    out_type=jax.ShapeDtypeStruct((N,), jnp.float32),
    scratch_types=[
        pltpu.VMEM((NR,), jnp.float32),
        pltpu.VMEM((P,), jnp.int32),
        pltpu.VMEM((P,), jnp.float32),
        pltpu.VMEM((P,), jnp.float32),
    ],
)
def _sc_gather(table_hbm, ri_hbm, cs_hbm, out_hbm, tbl, riv, csv, ov):
    wid = lax.axis_index("s") * 2 + lax.axis_index("c")
    base = wid * CH
    pltpu.sync_copy(table_hbm, tbl)

    def piece(p, c):
        gb = base + p * P
        pltpu.sync_copy(ri_hbm.at[pl.ds(gb, P)], riv)
        pltpu.sync_copy(cs_hbm.at[pl.ds(gb, P)], csv)

        @plsc.parallel_loop(0, P // 16, 1, unroll=8)
        def _inner(j):
            off = j * 16
            idx = jnp.minimum(riv[pl.ds(off, 16)], NR - 1)
            bv = plsc.load_gather(tbl, [idx])
            ov[pl.ds(off, 16)] = jnp.exp(csv[pl.ds(off, 16)] - bv)

        pltpu.sync_copy(ov, out_hbm.at[pl.ds(gb, P)])
        return c

    lax.fori_loop(0, NP, piece, 0)


def kernel(alphas, ray_indices, n_rays):
    tables = _sc_boundary(ray_indices)
    pos = _sc_merge(tables)
    a3 = alphas.reshape(G, ROWS, 128)
    cs = _tc_scan(a3).reshape(-1)
    table = _sc_posval(pos, cs)
    return _sc_gather(table, ray_indices, cs)


# speedup vs baseline: 256.3949x; 1.0226x over previous
import functools

import jax
import jax.numpy as jnp
from jax import lax
from jax.experimental import pallas as pl
from jax.experimental.pallas import tpu as pltpu
from jax.experimental.pallas import tpu_sc as plsc

N = 4194304
NR = 65536
G = 2
SUB = 128
ROWS = SUB * 128
NT = 32
CH = N // NT
P = 16384
NP = CH // P
PB = 32768
NPB = CH // PB


def _scan_body(a_ref, o_ref, lv_ref, w1_ref, w2_ref, off2_ref, s1last_ref,
               w3run_ref, off3_ref, s2last_ref):
    g = pl.program_id(0)

    @pl.when(g == 0)
    def _init():
        zero = jnp.zeros((1, 1), jnp.float32)
        s1last_ref[...] = zero
        w3run_ref[...] = zero
        off3_ref[...] = zero
        s2last_ref[...] = zero

    lv_ref[...] = jnp.log1p(-a_ref[0]).T.reshape(128, SUB, 128)

    acc = lv_ref[0]
    w1_ref[0] = acc
    for j in range(1, 128):
        acc = acc + lv_ref[j]
        w1_ref[j] = acc
    s1 = acc

    acc2 = s1[:, 0:1]
    w2_ref[:, 0:1] = acc2
    for l in range(1, 128):
        acc2 = acc2 + s1[:, l:l + 1]
        w2_ref[:, l:l + 1] = acc2
    w2 = w2_ref[...]

    s2col = w2[:, 127:128]
    w3run = w3run_ref[...]
    off3 = off3_ref[...]
    s2last = s2last_ref[...]
    for s in range(SUB):
        off2_ref[s:s + 1, :] = s2last
        w3run = w3run + s2col[s:s + 1, :]
        s2last = w3run + off3
        kg = g * SUB + s
        pred = (kg + 1) % 128 == 0
        off3 = jnp.where(pred, off3 + w3run, off3)
        w3run = jnp.where(pred, jnp.zeros((1, 1), jnp.float32), w3run)
    w3run_ref[...] = w3run
    off3_ref[...] = off3
    s2last_ref[...] = s2last

    s1f = w2 + off2_ref[...]
    lastcol = s1f[:, 127:128]
    shifted = jnp.concatenate([s1last_ref[...], lastcol[:SUB - 1]], axis=0)
    off1 = jnp.concatenate([shifted, s1f[:, :127]], axis=1)
    s1last_ref[...] = lastcol[SUB - 1:SUB]

    cs3 = (w1_ref[...] + off1[None]) - lv_ref[...]
    o_ref[0] = cs3.reshape(128, ROWS).T


def _tc_scan(a4):
    return pl.pallas_call(
        _scan_body,
        grid=(G,),
        in_specs=[pl.BlockSpec((1, ROWS, 128), lambda i: (i, 0, 0))],
        out_specs=pl.BlockSpec((1, ROWS, 128), lambda i: (i, 0, 0)),
        out_shape=jax.ShapeDtypeStruct((G, ROWS, 128), jnp.float32),
        scratch_shapes=[
            pltpu.VMEM((128, SUB, 128), jnp.float32),
            pltpu.VMEM((128, SUB, 128), jnp.float32),
            pltpu.VMEM((SUB, 128), jnp.float32),
            pltpu.VMEM((SUB, 1), jnp.float32),
            pltpu.VMEM((1, 1), jnp.float32),
            pltpu.VMEM((1, 1), jnp.float32),
            pltpu.VMEM((1, 1), jnp.float32),
            pltpu.VMEM((1, 1), jnp.float32),
        ],
    )(a4)


_MESH = plsc.VectorSubcoreMesh(core_axis_name="c", subcore_axis_name="s")


@functools.partial(
    pl.kernel,
    mesh=_MESH,
    compiler_params=pltpu.CompilerParams(needs_layout_passes=False),
    out_type=jax.ShapeDtypeStruct((NT, NR), jnp.int32),
    scratch_types=[
        pltpu.VMEM((PB + 16,), jnp.int32),
        pltpu.VMEM((NR,), jnp.int32),
    ],
)
def _sc_boundary(ri_hbm, tables_hbm, riv, tbl):
    wid = lax.axis_index("s") * 2 + lax.axis_index("c")
    base = wid * CH

    @plsc.parallel_loop(0, NR // 16, 1, unroll=8)
    def _initb(i):
        tbl[pl.ds(i * 16, 16)] = jnp.full((16,), N - 1, jnp.int32)

    def piece(p, c):
        gb = base + p * PB
        pltpu.sync_copy(ri_hbm.at[pl.ds(gb, PB)], riv.at[pl.ds(16, PB)])

        @pl.when(gb == 0)
        def _first():
            riv[pl.ds(0, 16)] = jnp.full((16,), -1, jnp.int32)

        @pl.when(gb > 0)
        def _rest():
            pltpu.sync_copy(ri_hbm.at[pl.ds(gb - 16, 16)], riv.at[pl.ds(0, 16)])

        @plsc.parallel_loop(0, PB // 16, 1, unroll=16)
        def _inner(j):
            off = j * 16
            cur = riv[pl.ds(off + 16, 16)]
            idxv = lax.iota(jnp.int32, 16) + (off + 15)
            prev = plsc.load_gather(riv, [idxv])
            bnd = cur != prev
            cur = jnp.minimum(cur, NR - 1)
            pos = lax.iota(jnp.int32, 16) + (gb + off)
            plsc.store_scatter(tbl, [cur], pos, mask=bnd)

        return c

    lax.fori_loop(0, NPB, piece, 0)
    pltpu.sync_copy(tbl, tables_hbm.at[wid])


@functools.partial(
    pl.kernel,
    mesh=_MESH,
    compiler_params=pltpu.CompilerParams(needs_layout_passes=False),
    out_type=jax.ShapeDtypeStruct((NR,), jnp.int32),
    scratch_types=[
        pltpu.VMEM((NT, NR // NT), jnp.int32),
        pltpu.SemaphoreType.DMA,
    ],
)
def _sc_merge(tables_hbm, table_hbm, stg, sem):
    wid = lax.axis_index("s") * 2 + lax.axis_index("c")
    s0 = wid * (NR // NT)
    copies = [
        pltpu.make_async_copy(tables_hbm.at[t, pl.ds(s0, NR // NT)], stg.at[t], sem)
        for t in range(NT)
    ]
    for c in copies:
        c.start()
    for c in copies:
        c.wait()

    @plsc.parallel_loop(0, NR // NT // 16, 1, unroll=4)
    def _mj(j):
        off = j * 16
        acc = stg[0, pl.ds(off, 16)]
        for t in range(1, NT):
            acc = jnp.minimum(acc, stg[t, pl.ds(off, 16)])
        stg[0, pl.ds(off, 16)] = acc
    pltpu.sync_copy(stg.at[0], table_hbm.at[pl.ds(s0, NR // NT)])


@functools.partial(
    pl.kernel,
    mesh=_MESH,
    compiler_params=pltpu.CompilerParams(needs_layout_passes=False),
    out_type=jax.ShapeDtypeStruct((NR,), jnp.float32),
    scratch_types=[
        pltpu.VMEM((NR // NT,), jnp.int32),
        pltpu.VMEM((NR // NT,), jnp.float32),
        pltpu.SemaphoreType.DMA,
    ],
)
def _sc_posval(pos_hbm, cs_hbm, table_hbm, posv, valv, sem):
    wid = lax.axis_index("s") * 2 + lax.axis_index("c")
    s0 = wid * (NR // NT)
    pltpu.sync_copy(pos_hbm.at[pl.ds(s0, NR // NT)], posv)
    pltpu.async_copy(cs_hbm.at[posv], valv, sem).wait()
    pltpu.sync_copy(valv, table_hbm.at[pl.ds(s0, NR // NT)])


@functools.partial(
    pl.kernel,
    mesh=_MESH,
    compiler_params=pltpu.CompilerParams(needs_layout_passes=False),
    out_type=jax.ShapeDtypeStruct((N,), jnp.float32),
    scratch_types=[
        pltpu.VMEM((NR,), jnp.float32),
        pltpu.VMEM((P,), jnp.int32),
        pltpu.VMEM((P,), jnp.float32),
        pltpu.VMEM((P,), jnp.float32),
    ],
)
def _sc_gather(table_hbm, ri_hbm, cs_hbm, out_hbm, tbl, riv, csv, ov):
    wid = lax.axis_index("s") * 2 + lax.axis_index("c")
    base = wid * CH
    pltpu.sync_copy(table_hbm, tbl)

    def piece(p, c):
        gb = base + p * P
        pltpu.sync_copy(ri_hbm.at[pl.ds(gb, P)], riv)
        pltpu.sync_copy(cs_hbm.at[pl.ds(gb, P)], csv)

        @plsc.parallel_loop(0, P // 16, 1, unroll=16)
        def _inner(j):
            off = j * 16
            idx = jnp.minimum(riv[pl.ds(off, 16)], NR - 1)
            bv = plsc.load_gather(tbl, [idx])
            ov[pl.ds(off, 16)] = jnp.exp(csv[pl.ds(off, 16)] - bv)

        pltpu.sync_copy(ov, out_hbm.at[pl.ds(gb, P)])
        return c

    lax.fori_loop(0, NP, piece, 0)


def kernel(alphas, ray_indices, n_rays):
    tables = _sc_boundary(ray_indices)
    pos = _sc_merge(tables)
    a3 = alphas.reshape(G, ROWS, 128)
    cs = _tc_scan(a3).reshape(-1)
    table = _sc_posval(pos, cs)
    return _sc_gather(table, ray_indices, cs)


# speedup vs baseline: 266.4953x; 1.0394x over previous
import functools

import jax
import jax.numpy as jnp
from jax import lax
from jax.experimental import pallas as pl
from jax.experimental.pallas import tpu as pltpu
from jax.experimental.pallas import tpu_sc as plsc

N = 4194304
NR = 65536
G = 2
SUB = 128
ROWS = SUB * 128
NT = 32
CH = N // NT
P = 16384
NP = CH // P
PB = 32768
NPB = CH // PB


def _scan_body(a_ref, o_ref, lv_ref, w1_ref, w2_ref, off2_ref, s1last_ref,
               w3run_ref, off3_ref, s2last_ref):
    g = pl.program_id(0)

    @pl.when(g == 0)
    def _init():
        zero = jnp.zeros((1, 1), jnp.float32)
        s1last_ref[...] = zero
        w3run_ref[...] = zero
        off3_ref[...] = zero
        s2last_ref[...] = zero

    lv_ref[...] = jnp.log1p(-a_ref[0]).T.reshape(128, SUB, 128)

    acc = lv_ref[0]
    w1_ref[0] = acc
    for j in range(1, 128):
        acc = acc + lv_ref[j]
        w1_ref[j] = acc
    s1 = acc

    acc2 = s1[:, 0:1]
    w2_ref[:, 0:1] = acc2
    for l in range(1, 128):
        acc2 = acc2 + s1[:, l:l + 1]
        w2_ref[:, l:l + 1] = acc2
    w2 = w2_ref[...]

    s2col = w2[:, 127:128]
    w3run = w3run_ref[...]
    off3 = off3_ref[...]
    s2last = s2last_ref[...]
    for s in range(SUB):
        off2_ref[s:s + 1, :] = s2last
        w3run = w3run + s2col[s:s + 1, :]
        s2last = w3run + off3
        kg = g * SUB + s
        pred = (kg + 1) % 128 == 0
        off3 = jnp.where(pred, off3 + w3run, off3)
        w3run = jnp.where(pred, jnp.zeros((1, 1), jnp.float32), w3run)
    w3run_ref[...] = w3run
    off3_ref[...] = off3
    s2last_ref[...] = s2last

    s1f = w2 + off2_ref[...]
    lastcol = s1f[:, 127:128]
    shifted = jnp.concatenate([s1last_ref[...], lastcol[:SUB - 1]], axis=0)
    off1 = jnp.concatenate([shifted, s1f[:, :127]], axis=1)
    s1last_ref[...] = lastcol[SUB - 1:SUB]

    cs3 = (w1_ref[...] + off1[None]) - lv_ref[...]
    o_ref[0] = cs3.reshape(128, ROWS).T


def _tc_scan(a4):
    return pl.pallas_call(
        _scan_body,
        grid=(G,),
        in_specs=[pl.BlockSpec((1, ROWS, 128), lambda i: (i, 0, 0))],
        out_specs=pl.BlockSpec((1, ROWS, 128), lambda i: (i, 0, 0)),
        out_shape=jax.ShapeDtypeStruct((G, ROWS, 128), jnp.float32),
        scratch_shapes=[
            pltpu.VMEM((128, SUB, 128), jnp.float32),
            pltpu.VMEM((128, SUB, 128), jnp.float32),
            pltpu.VMEM((SUB, 128), jnp.float32),
            pltpu.VMEM((SUB, 1), jnp.float32),
            pltpu.VMEM((1, 1), jnp.float32),
            pltpu.VMEM((1, 1), jnp.float32),
            pltpu.VMEM((1, 1), jnp.float32),
            pltpu.VMEM((1, 1), jnp.float32),
        ],
    )(a4)


_MESH = plsc.VectorSubcoreMesh(core_axis_name="c", subcore_axis_name="s")


@functools.partial(
    pl.kernel,
    mesh=_MESH,
    compiler_params=pltpu.CompilerParams(needs_layout_passes=False),
    out_type=jax.ShapeDtypeStruct((NT, NR), jnp.int32),
    scratch_types=[
        pltpu.VMEM((PB + 16,), jnp.int32),
        pltpu.VMEM((NR,), jnp.int32),
    ],
)
def _sc_boundary(ri_hbm, tables_hbm, riv, tbl):
    wid = lax.axis_index("s") * 2 + lax.axis_index("c")
    base = wid * CH

    @plsc.parallel_loop(0, NR // 16, 1, unroll=8)
    def _initb(i):
        tbl[pl.ds(i * 16, 16)] = jnp.full((16,), N - 1, jnp.int32)

    def piece(p, c):
        gb = base + p * PB
        pltpu.sync_copy(ri_hbm.at[pl.ds(gb, PB)], riv.at[pl.ds(16, PB)])

        @pl.when(gb == 0)
        def _first():
            riv[pl.ds(0, 16)] = jnp.full((16,), -1, jnp.int32)

        @pl.when(gb > 0)
        def _rest():
            pltpu.sync_copy(ri_hbm.at[pl.ds(gb - 16, 16)], riv.at[pl.ds(0, 16)])

        @plsc.parallel_loop(0, PB // 16, 1, unroll=16)
        def _inner(j):
            off = j * 16
            cur = riv[pl.ds(off + 16, 16)]
            idxv = lax.iota(jnp.int32, 16) + (off + 15)
            prev = plsc.load_gather(riv, [idxv])
            bnd = cur != prev
            cur = jnp.minimum(cur, NR - 1)
            pos = lax.iota(jnp.int32, 16) + (gb + off)
            plsc.store_scatter(tbl, [cur], pos, mask=bnd)

        return c

    lax.fori_loop(0, NPB, piece, 0)
    pltpu.sync_copy(tbl, tables_hbm.at[wid])


@functools.partial(
    pl.kernel,
    mesh=_MESH,
    compiler_params=pltpu.CompilerParams(needs_layout_passes=False),
    out_type=jax.ShapeDtypeStruct((NR,), jnp.int32),
    scratch_types=[
        pltpu.VMEM((NT, NR // NT), jnp.int32),
        pltpu.SemaphoreType.DMA,
    ],
)
def _sc_merge(tables_hbm, table_hbm, stg, sem):
    wid = lax.axis_index("s") * 2 + lax.axis_index("c")
    s0 = wid * (NR // NT)
    copies = [
        pltpu.make_async_copy(tables_hbm.at[t, pl.ds(s0, NR // NT)], stg.at[t], sem)
        for t in range(NT)
    ]
    for c in copies:
        c.start()
    for c in copies:
        c.wait()

    @plsc.parallel_loop(0, NR // NT // 16, 1, unroll=4)
    def _mj(j):
        off = j * 16
        acc = stg[0, pl.ds(off, 16)]
        for t in range(1, NT):
            acc = jnp.minimum(acc, stg[t, pl.ds(off, 16)])
        stg[0, pl.ds(off, 16)] = acc
    pltpu.sync_copy(stg.at[0], table_hbm.at[pl.ds(s0, NR // NT)])


@functools.partial(
    pl.kernel,
    mesh=_MESH,
    compiler_params=pltpu.CompilerParams(needs_layout_passes=False),
    out_type=jax.ShapeDtypeStruct((NR,), jnp.float32),
    scratch_types=[
        pltpu.VMEM((NR // NT,), jnp.int32),
        pltpu.VMEM((NR // NT,), jnp.float32),
        pltpu.SemaphoreType.DMA,
    ],
)
def _sc_posval(pos_hbm, cs_hbm, table_hbm, posv, valv, sem):
    wid = lax.axis_index("s") * 2 + lax.axis_index("c")
    s0 = wid * (NR // NT)
    pltpu.sync_copy(pos_hbm.at[pl.ds(s0, NR // NT)], posv)
    pltpu.async_copy(cs_hbm.at[posv], valv, sem).wait()
    pltpu.sync_copy(valv, table_hbm.at[pl.ds(s0, NR // NT)])


PG = 8192
NPG = CH // PG


@functools.partial(
    pl.kernel,
    mesh=_MESH,
    compiler_params=pltpu.CompilerParams(needs_layout_passes=False),
    out_type=jax.ShapeDtypeStruct((N,), jnp.float32),
    scratch_types=[
        pltpu.VMEM((NR,), jnp.float32),
        pltpu.VMEM((2, PG), jnp.int32),
        pltpu.VMEM((2, PG), jnp.float32),
        pltpu.VMEM((2, PG), jnp.float32),
        pltpu.SemaphoreType.DMA,
        pltpu.SemaphoreType.DMA,
        pltpu.SemaphoreType.DMA,
    ],
)
def _sc_gather(table_hbm, ri_hbm, cs_hbm, out_hbm, tbl, riv, csv, ov,
               sem_ri, sem_cs, sem_out):
    wid = lax.axis_index("s") * 2 + lax.axis_index("c")
    base = wid * CH
    pltpu.sync_copy(table_hbm, tbl)

    def in_copies(p, b):
        gb = base + p * PG
        return (
            pltpu.make_async_copy(ri_hbm.at[pl.ds(gb, PG)], riv.at[b], sem_ri),
            pltpu.make_async_copy(cs_hbm.at[pl.ds(gb, PG)], csv.at[b], sem_cs),
        )

    out_copies = [None] * NPG
    c_ri, c_cs = in_copies(0, 0)
    c_ri.start()
    c_cs.start()
    for p in range(NPG):
        b = p % 2
        c_ri.wait()
        c_cs.wait()
        if p + 1 < NPG:
            n_ri, n_cs = in_copies(p + 1, 1 - b)
            n_ri.start()
            n_cs.start()
        if p >= 2:
            out_copies[p - 2].wait()

        @plsc.parallel_loop(0, PG // 16, 1, unroll=16)
        def _inner(j):
            off = j * 16
            idx = jnp.minimum(riv[b, pl.ds(off, 16)], NR - 1)
            bv = plsc.load_gather(tbl, [idx])
            ov[b, pl.ds(off, 16)] = jnp.exp(csv[b, pl.ds(off, 16)] - bv)

        oc = pltpu.make_async_copy(
            ov.at[b], out_hbm.at[pl.ds(base + p * PG, PG)], sem_out)
        oc.start()
        out_copies[p] = oc
        if p + 1 < NPG:
            c_ri, c_cs = n_ri, n_cs
    out_copies[NPG - 2].wait()
    out_copies[NPG - 1].wait()


def kernel(alphas, ray_indices, n_rays):
    tables = _sc_boundary(ray_indices)
    pos = _sc_merge(tables)
    a3 = alphas.reshape(G, ROWS, 128)
    cs = _tc_scan(a3).reshape(-1)
    table = _sc_posval(pos, cs)
    return _sc_gather(table, ray_indices, cs)
